# Initial kernel scaffold; baseline (speedup 1.0000x reference)
import jax
import jax.numpy as jnp
from jax.experimental import pallas as pl


def kernel(adj_indices, adj_values, E_u_0, E_i_0, u_mul_s, v_mul_s, ut, vt):
    raise NotImplementedError("write your pallas kernel here")


# speedup vs baseline: 3.3094x; 3.3094x over previous
import functools

import jax
import jax.numpy as jnp
from jax import lax
from jax.experimental import pallas as pl
from jax.experimental.pallas import tpu as pltpu
from jax.experimental.pallas import tpu_sc as plsc

D = 64
CHUNK = 128
NTILES = 16
NCORES = 2
HALF = 25000
ACC_ROWS = 16 * 13 * 128
OUT_PER_TILE = 1560


def _spmm_body(nchunk, dst_hbm, src_hbm, val_hbm, table_hbm, out_hbm,
               idx_d, idx_s, vals_v, rows_v, sidx_v, acc_sh, sem):
    c = lax.axis_index("c")
    s = lax.axis_index("s")

    zero16 = jnp.zeros((16,), jnp.float32)

    def zrow(e, _):
        for j in range(D // 16):
            rows_v[e, pl.ds(16 * j, 16)] = zero16
        return _

    lax.fori_loop(0, CHUNK, zrow, 0)
    for k in range(ACC_ROWS // (NTILES * CHUNK)):
        pltpu.sync_copy(rows_v, acc_sh.at[pl.ds((s * 13 + k) * CHUNK, CHUNK)])
    plsc.subcore_barrier()

    def chunk_body(ci, _):
        base = (s * nchunk + ci) * CHUNK
        pltpu.sync_copy(src_hbm.at[pl.ds(base, CHUNK)], idx_s)
        gather = pltpu.async_copy(table_hbm.at[idx_s], rows_v, sem)
        pltpu.sync_copy(dst_hbm.at[pl.ds(base, CHUNK)], idx_d)
        pltpu.sync_copy(val_hbm.at[pl.ds(base, CHUNK)], vals_v)
        for j in range(CHUNK // 16):
            d16 = idx_d[pl.ds(16 * j, 16)]
            loc = d16 - c * HALF
            ok = (loc >= 0) & (loc < HALF)
            sidx_v[pl.ds(16 * j, 16)] = jnp.where(ok, loc, HALF)
        gather.wait()
        for j in range(CHUNK // 16):
            vv = vals_v[pl.ds(16 * j, 16)]
            for t in range(16):
                e = 16 * j + t
                v = vv[t]
                for q in range(D // 16):
                    rows_v[e, pl.ds(16 * q, 16)] = (
                        rows_v[e, pl.ds(16 * q, 16)] * v)
        pltpu.sync_copy(rows_v, acc_sh.at[sidx_v], add=True)
        return _

    lax.fori_loop(0, nchunk, chunk_body, 0)
    plsc.subcore_barrier()

    @pl.when(s < NTILES - 1)
    def _():
        pltpu.sync_copy(
            acc_sh.at[pl.ds(s * OUT_PER_TILE, OUT_PER_TILE)],
            out_hbm.at[pl.ds(c * HALF + s * OUT_PER_TILE, OUT_PER_TILE)])

    @pl.when(s == NTILES - 1)
    def _():
        base = (NTILES - 1) * OUT_PER_TILE
        last = HALF - base
        pltpu.sync_copy(
            acc_sh.at[pl.ds(base, last)],
            out_hbm.at[pl.ds(c * HALF + base, last)])


@functools.cache
def _make_spmm(n_rows, nchunk):
    return pl.kernel(
        functools.partial(_spmm_body, nchunk),
        out_type=jax.ShapeDtypeStruct((n_rows, D), jnp.float32),
        mesh=plsc.VectorSubcoreMesh(core_axis_name="c", subcore_axis_name="s"),
        compiler_params=pltpu.CompilerParams(use_tc_tiling_on_sc=False),
        scratch_types=[
            pltpu.VMEM((CHUNK,), jnp.int32),
            pltpu.VMEM((CHUNK,), jnp.int32),
            pltpu.VMEM((CHUNK,), jnp.float32),
            pltpu.VMEM((CHUNK, D), jnp.float32),
            pltpu.VMEM((CHUNK,), jnp.int32),
            pltpu.VMEM_SHARED((ACC_ROWS, D), jnp.float32),
            pltpu.SemaphoreType.DMA,
        ],
    )


def _sums_body(e0u, z1u, z2u, z3u, e0i, z1i, z2i, z3i,
               eu, su, ei, si):
    pu = e0u[...] + z1u[...] + z2u[...]
    su[...] = pu
    eu[...] = pu + z3u[...]
    pi = e0i[...] + z1i[...] + z2i[...]
    si[...] = pi
    ei[...] = pi + z3i[...]


def _kt_body(vtp, s_i, pu):
    pu[...] = jnp.dot(vtp[...], s_i[...], preferred_element_type=jnp.float32)


def _g_body(e0u, up, pu, e0i, vp, qi, gu, gi):
    gu[...] = e0u[...] + jnp.dot(up[...], pu[...],
                                 preferred_element_type=jnp.float32)
    gi[...] = e0i[...] + jnp.dot(vp[...], qi[...],
                                 preferred_element_type=jnp.float32)


def kernel(adj_indices, adj_values, E_u_0, E_i_0, u_mul_s, v_mul_s, ut, vt):
    n_users, _ = E_u_0.shape
    n_items, _ = E_i_0.shape
    nnz = adj_values.shape[0]
    row = adj_indices[0].astype(jnp.int32)
    col = adj_indices[1].astype(jnp.int32)
    val = adj_values.astype(jnp.float32)

    grp = NTILES * CHUNK
    epad = -(-nnz // grp) * grp
    pad = epad - nnz
    if pad:
        row = jnp.concatenate([row, jnp.zeros((pad,), jnp.int32)])
        col = jnp.concatenate([col, jnp.zeros((pad,), jnp.int32)])
        val = jnp.concatenate([val, jnp.zeros((pad,), jnp.float32)])
    nchunk = epad // grp

    spmm_u = _make_spmm(n_users, nchunk)
    spmm_i = _make_spmm(n_items, nchunk)

    Zu1 = spmm_u(row, col, val, E_i_0)
    Zi1 = spmm_i(col, row, val, E_u_0)
    Zu2 = spmm_u(row, col, val, Zi1)
    Zi2 = spmm_i(col, row, val, Zu1)
    Zu3 = spmm_u(row, col, val, Zi2)
    Zi3 = spmm_i(col, row, val, Zu2)

    nb = 50
    bu = n_users // nb
    bi = n_items // nb
    blk_u = pl.BlockSpec((bu, D), lambda i: (i, 0))
    blk_i = pl.BlockSpec((bi, D), lambda i: (i, 0))
    E_u, S_u, E_i, S_i = pl.pallas_call(
        _sums_body,
        grid=(nb,),
        in_specs=[blk_u] * 4 + [blk_i] * 4,
        out_specs=[blk_u, blk_u, blk_i, blk_i],
        out_shape=[jax.ShapeDtypeStruct((n_users, D), jnp.float32)] * 2
        + [jax.ShapeDtypeStruct((n_items, D), jnp.float32)] * 2,
    )(E_u_0, Zu1, Zu2, Zu3, E_i_0, Zi1, Zi2, Zi3)

    q = ut.shape[0]
    qp = 32
    vtp = jnp.pad(vt, ((0, qp - q), (0, 0)))
    utp = jnp.pad(ut, ((0, qp - q), (0, 0)))
    up = jnp.pad(u_mul_s, ((0, 0), (0, qp - q)))
    vp = jnp.pad(v_mul_s, ((0, 0), (0, qp - q)))

    P_u = pl.pallas_call(
        _kt_body,
        out_shape=jax.ShapeDtypeStruct((qp, D), jnp.float32),
    )(vtp, S_i)
    Q_i = pl.pallas_call(
        _kt_body,
        out_shape=jax.ShapeDtypeStruct((qp, D), jnp.float32),
    )(utp, S_u)

    G_u, G_i = pl.pallas_call(
        _g_body,
        grid=(nb,),
        in_specs=[
            blk_u,
            pl.BlockSpec((bu, qp), lambda i: (i, 0)),
            pl.BlockSpec((qp, D), lambda i: (0, 0)),
            blk_i,
            pl.BlockSpec((bi, qp), lambda i: (i, 0)),
            pl.BlockSpec((qp, D), lambda i: (0, 0)),
        ],
        out_specs=[blk_u, blk_i],
        out_shape=[jax.ShapeDtypeStruct((n_users, D), jnp.float32),
                   jax.ShapeDtypeStruct((n_items, D), jnp.float32)],
    )(E_u_0, up, P_u, E_i_0, vp, Q_i)

    return (E_u, E_i, G_u, G_i)


# speedup vs baseline: 4.3976x; 1.3288x over previous
import functools

import jax
import jax.numpy as jnp
from jax import lax
from jax.experimental import pallas as pl
from jax.experimental.pallas import tpu as pltpu
from jax.experimental.pallas import tpu_sc as plsc

D = 64
CHUNK = 128
NTILES = 16
NCORES = 2
HALF = 25000
ACC_ROWS = 16 * 13 * 128
OUT_PER_TILE = 1560


def _spmm_body(nchunk, pk_hbm, table_hbm, out_hbm,
               pkb0, pkb1, rows0, rows1, sidx0, sidx1, acc_sh,
               si0, si1, sg0, sg1, ss0, ss1):
    c = lax.axis_index("c")
    s = lax.axis_index("s")
    pkb = (pkb0, pkb1)
    rows = (rows0, rows1)
    sidx = (sidx0, sidx1)
    si = (si0, si1)
    sg = (sg0, sg1)
    ss = (ss0, ss1)

    zero16 = jnp.zeros((16,), jnp.float32)

    def zrow(e, _):
        for j in range(D // 16):
            rows0[e, pl.ds(16 * j, 16)] = zero16
        return _

    lax.fori_loop(0, CHUNK, zrow, 0)
    for k in range(ACC_ROWS // (NTILES * CHUNK)):
        pltpu.sync_copy(rows0, acc_sh.at[pl.ds((s * 13 + k) * CHUNK, CHUNK)])
    plsc.subcore_barrier()

    base_cid = s * nchunk

    def start_idx(ci, b):
        pltpu.async_copy(pk_hbm.at[base_cid + ci], pkb[b], si[b])

    def wait_idx(ci, b):
        pltpu.make_async_copy(pk_hbm.at[base_cid + ci], pkb[b], si[b]).wait()

    def start_gather(b):
        pltpu.async_copy(table_hbm.at[pkb[b].at[1]], rows[b], sg[b])

    def wait_gather(b):
        pltpu.make_async_copy(table_hbm.at[pkb[b].at[1]], rows[b],
                              sg[b]).wait()

    def start_scatter(b):
        pltpu.async_copy(rows[b], acc_sh.at[sidx[b]], ss[b], add=True)

    def wait_scatter(b):
        pltpu.make_async_copy(rows[b], acc_sh.at[sidx[b]], ss[b]).wait()

    def compute(b):
        for j in range(CHUNK // 16):
            d16 = pkb[b][0, pl.ds(16 * j, 16)]
            loc = d16 - c * HALF
            ok = (loc >= 0) & (loc < HALF)
            sidx[b][pl.ds(16 * j, 16)] = jnp.where(ok, loc, HALF)
        for j in range(CHUNK // 16):
            vv = plsc.bitcast(pkb[b][2, pl.ds(16 * j, 16)], jnp.float32)
            for t in range(16):
                e = 16 * j + t
                v = vv[t]
                for q in range(D // 16):
                    rows[b][e, pl.ds(16 * q, 16)] = (
                        rows[b][e, pl.ds(16 * q, 16)] * v)

    start_idx(0, 0)
    start_idx(1, 1)
    wait_idx(0, 0)
    start_gather(0)

    def pair_body(ci2, carry):
        for b in (0, 1):
            ci = ci2 * 2 + b
            nb = 1 - b

            @pl.when((ci >= 1) & (ci + 1 < nchunk))
            def _():
                wait_scatter(nb)

            @pl.when(ci + 1 < nchunk)
            def _():
                wait_idx(ci + 1, nb)
                start_gather(nb)

            wait_gather(b)
            compute(b)

            @pl.when(ci + 2 < nchunk)
            def _():
                start_idx(ci + 2, b)

            start_scatter(b)
        return carry

    lax.fori_loop(0, nchunk // 2, pair_body, 0)
    wait_scatter(0)
    wait_scatter(1)
    plsc.subcore_barrier()

    @pl.when(s < NTILES - 1)
    def _():
        pltpu.sync_copy(
            acc_sh.at[pl.ds(s * OUT_PER_TILE, OUT_PER_TILE)],
            out_hbm.at[pl.ds(c * HALF + s * OUT_PER_TILE, OUT_PER_TILE)])

    @pl.when(s == NTILES - 1)
    def _():
        base = (NTILES - 1) * OUT_PER_TILE
        last = HALF - base
        pltpu.sync_copy(
            acc_sh.at[pl.ds(base, last)],
            out_hbm.at[pl.ds(c * HALF + base, last)])


@functools.cache
def _make_spmm(n_rows, nchunk):
    return pl.kernel(
        functools.partial(_spmm_body, nchunk),
        out_type=jax.ShapeDtypeStruct((n_rows, D), jnp.float32),
        mesh=plsc.VectorSubcoreMesh(core_axis_name="c", subcore_axis_name="s"),
        compiler_params=pltpu.CompilerParams(use_tc_tiling_on_sc=False,
                                             needs_layout_passes=False),
        scratch_types=[
            pltpu.VMEM((3, CHUNK), jnp.int32),
            pltpu.VMEM((3, CHUNK), jnp.int32),
            pltpu.VMEM((CHUNK, D), jnp.float32),
            pltpu.VMEM((CHUNK, D), jnp.float32),
            pltpu.VMEM((CHUNK,), jnp.int32),
            pltpu.VMEM((CHUNK,), jnp.int32),
            pltpu.VMEM_SHARED((ACC_ROWS, D), jnp.float32),
            pltpu.SemaphoreType.DMA,
            pltpu.SemaphoreType.DMA,
            pltpu.SemaphoreType.DMA,
            pltpu.SemaphoreType.DMA,
            pltpu.SemaphoreType.DMA,
            pltpu.SemaphoreType.DMA,
        ],
    )


def _sums_body(e0u, z1u, z2u, z3u, e0i, z1i, z2i, z3i,
               eu, su, ei, si):
    pu = e0u[...] + z1u[...] + z2u[...]
    su[...] = pu
    eu[...] = pu + z3u[...]
    pi = e0i[...] + z1i[...] + z2i[...]
    si[...] = pi
    ei[...] = pi + z3i[...]


def _kt_body(vtp, s_i, pu):
    pu[...] = jnp.dot(vtp[...], s_i[...], preferred_element_type=jnp.float32)


def _g_body(e0u, up, pu, e0i, vp, qi, gu, gi):
    gu[...] = e0u[...] + jnp.dot(up[...], pu[...],
                                 preferred_element_type=jnp.float32)
    gi[...] = e0i[...] + jnp.dot(vp[...], qi[...],
                                 preferred_element_type=jnp.float32)


def kernel(adj_indices, adj_values, E_u_0, E_i_0, u_mul_s, v_mul_s, ut, vt):
    n_users, _ = E_u_0.shape
    n_items, _ = E_i_0.shape
    nnz = adj_values.shape[0]
    row = adj_indices[0].astype(jnp.int32)
    col = adj_indices[1].astype(jnp.int32)
    val = adj_values.astype(jnp.float32)

    grp = NTILES * CHUNK * 2
    epad = -(-nnz // grp) * grp
    pad = epad - nnz
    if pad:
        row = jnp.concatenate([row, jnp.zeros((pad,), jnp.int32)])
        col = jnp.concatenate([col, jnp.zeros((pad,), jnp.int32)])
        val = jnp.concatenate([val, jnp.zeros((pad,), jnp.float32)])
    nchunk = epad // grp * 2

    vbits = jax.lax.bitcast_convert_type(val, jnp.int32)
    pk_u = jnp.stack([row, col, vbits]).reshape(3, -1, CHUNK).transpose(1, 0, 2)
    pk_i = jnp.stack([col, row, vbits]).reshape(3, -1, CHUNK).transpose(1, 0, 2)

    spmm_u = _make_spmm(n_users, nchunk)
    spmm_i = _make_spmm(n_items, nchunk)

    Zu1 = spmm_u(pk_u, E_i_0)
    Zi1 = spmm_i(pk_i, E_u_0)
    Zu2 = spmm_u(pk_u, Zi1)
    Zi2 = spmm_i(pk_i, Zu1)
    Zu3 = spmm_u(pk_u, Zi2)
    Zi3 = spmm_i(pk_i, Zu2)

    nb = 50
    bu = n_users // nb
    bi = n_items // nb
    blk_u = pl.BlockSpec((bu, D), lambda i: (i, 0))
    blk_i = pl.BlockSpec((bi, D), lambda i: (i, 0))
    E_u, S_u, E_i, S_i = pl.pallas_call(
        _sums_body,
        grid=(nb,),
        in_specs=[blk_u] * 4 + [blk_i] * 4,
        out_specs=[blk_u, blk_u, blk_i, blk_i],
        out_shape=[jax.ShapeDtypeStruct((n_users, D), jnp.float32)] * 2
        + [jax.ShapeDtypeStruct((n_items, D), jnp.float32)] * 2,
    )(E_u_0, Zu1, Zu2, Zu3, E_i_0, Zi1, Zi2, Zi3)

    q = ut.shape[0]
    qp = 32
    vtp = jnp.pad(vt, ((0, qp - q), (0, 0)))
    utp = jnp.pad(ut, ((0, qp - q), (0, 0)))
    up = jnp.pad(u_mul_s, ((0, 0), (0, qp - q)))
    vp = jnp.pad(v_mul_s, ((0, 0), (0, qp - q)))

    P_u = pl.pallas_call(
        _kt_body,
        out_shape=jax.ShapeDtypeStruct((qp, D), jnp.float32),
    )(vtp, S_i)
    Q_i = pl.pallas_call(
        _kt_body,
        out_shape=jax.ShapeDtypeStruct((qp, D), jnp.float32),
    )(utp, S_u)

    G_u, G_i = pl.pallas_call(
        _g_body,
        grid=(nb,),
        in_specs=[
            blk_u,
            pl.BlockSpec((bu, qp), lambda i: (i, 0)),
            pl.BlockSpec((qp, D), lambda i: (0, 0)),
            blk_i,
            pl.BlockSpec((bi, qp), lambda i: (i, 0)),
            pl.BlockSpec((qp, D), lambda i: (0, 0)),
        ],
        out_specs=[blk_u, blk_i],
        out_shape=[jax.ShapeDtypeStruct((n_users, D), jnp.float32),
                   jax.ShapeDtypeStruct((n_items, D), jnp.float32)],
    )(E_u_0, up, P_u, E_i_0, vp, Q_i)

    return (E_u, E_i, G_u, G_i)
